# Initial kernel scaffold; baseline (speedup 1.0000x reference)
import jax
import jax.numpy as jnp
from jax import lax
from jax.experimental import pallas as pl
from jax.experimental.pallas import tpu as pltpu
from jax.experimental.pallas import tpu_sc as plsc

N_NODES = 50000
NUM_SEGS = 512
DIM = 256
NC, NS = 2, 16
NW = NC * NS
TILE = 128
FULL_TILES = N_NODES // TILE
TAIL = N_NODES - FULL_TILES * TILE
TILES_PER_W = -(-FULL_TILES // NW)
CNTW = 16
ROWS_PER_SUB = NUM_SEGS // NS


def _sc_pool(nodes_hbm, ids_hbm, sums_hbm, cnts_hbm,
             rows_v, idx_v, ones_v, idx_t, zero_v, zcnt_v, acc_sh, cnt_sh):
    cid = lax.axis_index("c")
    sid = lax.axis_index("s")
    wid = sid * NC + cid

    def _zrow(i, _):
        def _zcol(j, _):
            zero_v[i, pl.ds(j * 16, 16)] = jnp.zeros((16,), jnp.float32)
            return 0
        return lax.fori_loop(0, DIM // 16, _zcol, 0)
    lax.fori_loop(0, ROWS_PER_SUB, _zrow, 0)

    def _orow(i, _):
        ones_v[i, :] = jnp.ones((CNTW,), jnp.float32)
        return 0
    lax.fori_loop(0, TILE, _orow, 0)

    def _zcrow(i, _):
        zcnt_v[i, :] = jnp.zeros((CNTW,), jnp.float32)
        return 0
    lax.fori_loop(0, ROWS_PER_SUB, _zcrow, 0)

    pltpu.sync_copy(zero_v, acc_sh.at[pl.ds(sid * ROWS_PER_SUB, ROWS_PER_SUB)])
    pltpu.sync_copy(zcnt_v, cnt_sh.at[pl.ds(sid * ROWS_PER_SUB, ROWS_PER_SUB)])
    plsc.subcore_barrier()

    for i in range(TILES_PER_W):
        t = wid + i * NW

        @pl.when(t < FULL_TILES)
        def _():
            base = t * TILE
            pltpu.sync_copy(ids_hbm.at[pl.ds(base, TILE)], idx_v)
            pltpu.sync_copy(nodes_hbm.at[pl.ds(base, TILE)], rows_v)
            pltpu.sync_copy(rows_v, acc_sh.at[idx_v], add=True)
            pltpu.sync_copy(ones_v, cnt_sh.at[idx_v], add=True)

    @pl.when(wid == NW - 1)
    def _():
        base = FULL_TILES * TILE
        pltpu.sync_copy(ids_hbm.at[pl.ds(base, TAIL)], idx_t)
        pltpu.sync_copy(nodes_hbm.at[pl.ds(base, TAIL)], rows_v.at[pl.ds(0, TAIL)])
        pltpu.sync_copy(rows_v.at[pl.ds(0, TAIL)], acc_sh.at[idx_t], add=True)
        pltpu.sync_copy(ones_v.at[pl.ds(0, TAIL)], cnt_sh.at[idx_t], add=True)

    plsc.subcore_barrier()

    r0 = sid * ROWS_PER_SUB
    pltpu.sync_copy(acc_sh.at[pl.ds(r0, ROWS_PER_SUB)],
                    sums_hbm.at[cid, pl.ds(r0, ROWS_PER_SUB)])
    pltpu.sync_copy(cnt_sh.at[pl.ds(r0, ROWS_PER_SUB)],
                    cnts_hbm.at[cid, pl.ds(r0, ROWS_PER_SUB)])


def _tc_finish(ps_ref, pc_ref, w1_ref, b1_ref, w2_ref, b2_ref, g_ref, z_ref):
    sums = ps_ref[0] + ps_ref[1]
    counts = jnp.sum(pc_ref[0] + pc_ref[1], axis=1, keepdims=True)
    g = sums / jnp.maximum(counts, 1.0)
    g_ref[...] = g
    h = lax.dot_general(g, w1_ref[...], (((1,), (1,)), ((), ())),
                        preferred_element_type=jnp.float32) + b1_ref[...]
    h = jnp.maximum(h, 0.0)
    z_ref[...] = lax.dot_general(h, w2_ref[...], (((1,), (1,)), ((), ())),
                                 preferred_element_type=jnp.float32) + b2_ref[...]


def kernel(node_rep, batch_ids, W1, b1, W2, b2):
    ids32 = batch_ids.astype(jnp.int32)

    mesh = plsc.VectorSubcoreMesh(core_axis_name="c", subcore_axis_name="s",
                                  num_cores=NC, num_subcores=NS)
    sums, cnts = pl.kernel(
        _sc_pool,
        out_type=(jax.ShapeDtypeStruct((NC, NUM_SEGS, DIM), jnp.float32),
                  jax.ShapeDtypeStruct((NC, NUM_SEGS, CNTW), jnp.float32)),
        mesh=mesh,
        scratch_types=[
            pltpu.VMEM((TILE, DIM), jnp.float32),
            pltpu.VMEM((TILE,), jnp.int32),
            pltpu.VMEM((TILE, CNTW), jnp.float32),
            pltpu.VMEM((TAIL,), jnp.int32),
            pltpu.VMEM((ROWS_PER_SUB, DIM), jnp.float32),
            pltpu.VMEM((ROWS_PER_SUB, CNTW), jnp.float32),
            pltpu.VMEM_SHARED((NUM_SEGS, DIM), jnp.float32),
            pltpu.VMEM_SHARED((NUM_SEGS, CNTW), jnp.float32),
        ],
    )(node_rep, ids32)

    g, z = pl.pallas_call(
        _tc_finish,
        out_shape=(jax.ShapeDtypeStruct((NUM_SEGS, DIM), jnp.float32),
                   jax.ShapeDtypeStruct((NUM_SEGS, DIM), jnp.float32)),
    )(sums, cnts, W1, b1.reshape(1, DIM), W2, b2.reshape(1, DIM))

    return (g, z)


# speedup vs baseline: 2.2309x; 2.2309x over previous
import jax
import jax.numpy as jnp
from jax import lax
from jax.experimental import pallas as pl
from jax.experimental.pallas import tpu as pltpu
from jax.experimental.pallas import tpu_sc as plsc

N_NODES = 50000
NUM_SEGS = 512
DIM = 256
SPAD = 544
NC, NS = 2, 16
SEGW = NUM_SEGS // NS
COLW = DIM // NC
T = 128
TK = 1024
NBLK = -(-N_NODES // TK)


def _tc_starts(ids_ref, out_ref):
    pid = pl.program_id(0)

    @pl.when(pid == 0)
    def _():
        out_ref[...] = jnp.zeros((1, SPAD), jnp.int32)

    rows = lax.broadcasted_iota(jnp.int32, (TK, 1), 0) + pid * TK
    segs = lax.broadcasted_iota(jnp.int32, (1, SPAD), 1)
    m = (ids_ref[...] < segs) & (rows < N_NODES)
    out_ref[...] += jnp.sum(m.astype(jnp.int32), axis=0, keepdims=True)


def _sc_pool(nodes_hbm, starts_hbm, sums_hbm, cnts_hbm, buf_v, acc_v, st_s, cnt_v):
    sid = lax.axis_index("s")
    cid = lax.axis_index("c")
    seg0 = sid * SEGW
    col0 = cid * COLW

    pltpu.sync_copy(starts_hbm, st_s)

    def _z(i, _):
        def _zc(j, _):
            acc_v[i, pl.ds(j * 16, 16)] = jnp.zeros((16,), jnp.float32)
            return 0
        return lax.fori_loop(0, COLW // 16, _zc, 0)
    lax.fori_loop(0, SEGW, _z, 0)

    def _seg(k, _):
        a = st_s[pl.ds(seg0 + k, 16)][0]
        b = st_s[pl.ds(seg0 + k + 1, 16)][0]
        t0 = pl.multiple_of((a // 8) * 8, 8)
        n_tiles = (b - t0 + T - 1) // T

        def _tile(j, _):
            t = t0 + j * T
            t_clamped = pl.multiple_of(jnp.minimum(t, N_NODES - T), 8)
            pltpu.sync_copy(
                nodes_hbm.at[pl.ds(t_clamped, T), pl.ds(col0, COLW)], buf_v)
            lo = jnp.maximum(a, t) - t_clamped
            hi = jnp.minimum(b, t_clamped + T) - t_clamped

            def _row(r, _):
                def _cols(j2, _):
                    plsc.addupdate(acc_v.at[k, pl.ds(j2 * 16, 16)],
                                   buf_v[r, pl.ds(j2 * 16, 16)])
                    return 0
                return lax.fori_loop(0, COLW // 16, _cols, 0)
            lax.fori_loop(lo, hi, _row, 0)
            return 0

        lax.fori_loop(0, n_tiles, _tile, 0)
        return 0

    lax.fori_loop(0, SEGW, _seg, 0)

    pltpu.sync_copy(acc_v, sums_hbm.at[pl.ds(seg0, SEGW), pl.ds(col0, COLW)])

    @pl.when(cid == 0)
    def _():
        for v in range(SEGW // 16):
            lo16 = st_s[pl.ds(seg0 + v * 16, 16)]
            hi16 = st_s[pl.ds(seg0 + v * 16 + 1, 16)]
            cnt_v[pl.ds(v * 16, 16)] = hi16 - lo16
        pltpu.sync_copy(cnt_v, cnts_hbm.at[pl.ds(seg0, SEGW)])


def _tc_finish(ps_ref, pc_ref, w1_ref, b1_ref, w2_ref, b2_ref, g_ref, z_ref):
    counts = jnp.maximum(pc_ref[...].astype(jnp.float32), 1.0)
    g = ps_ref[...] / counts
    g_ref[...] = g
    h = lax.dot_general(g, w1_ref[...], (((1,), (1,)), ((), ())),
                        preferred_element_type=jnp.float32) + b1_ref[...]
    h = jnp.maximum(h, 0.0)
    z_ref[...] = lax.dot_general(h, w2_ref[...], (((1,), (1,)), ((), ())),
                                 preferred_element_type=jnp.float32) + b2_ref[...]


def kernel(node_rep, batch_ids, W1, b1, W2, b2):
    ids32 = batch_ids.astype(jnp.int32).reshape(N_NODES, 1)

    starts2d = pl.pallas_call(
        _tc_starts,
        grid=(NBLK,),
        in_specs=[pl.BlockSpec((TK, 1), lambda i: (i, 0))],
        out_specs=pl.BlockSpec((1, SPAD), lambda i: (0, 0)),
        out_shape=jax.ShapeDtypeStruct((1, SPAD), jnp.int32),
    )(ids32)
    starts = starts2d.reshape(SPAD)

    mesh = plsc.VectorSubcoreMesh(core_axis_name="c", subcore_axis_name="s",
                                  num_cores=NC, num_subcores=NS)
    sums, cnts = pl.kernel(
        _sc_pool,
        out_type=(jax.ShapeDtypeStruct((NUM_SEGS, DIM), jnp.float32),
                  jax.ShapeDtypeStruct((NUM_SEGS,), jnp.int32)),
        mesh=mesh,
        scratch_types=[
            pltpu.VMEM((T, COLW), jnp.float32),
            pltpu.VMEM((SEGW, COLW), jnp.float32),
            pltpu.VMEM((SPAD,), jnp.int32),
            pltpu.VMEM((SEGW,), jnp.int32),
        ],
    )(node_rep, starts)

    g, z = pl.pallas_call(
        _tc_finish,
        out_shape=(jax.ShapeDtypeStruct((NUM_SEGS, DIM), jnp.float32),
                   jax.ShapeDtypeStruct((NUM_SEGS, DIM), jnp.float32)),
    )(sums, cnts.reshape(NUM_SEGS, 1), W1, b1.reshape(1, DIM),
      W2, b2.reshape(1, DIM))

    return (g, z)


# speedup vs baseline: 3.5025x; 1.5700x over previous
import jax
import jax.numpy as jnp
from jax import lax
from jax.experimental import pallas as pl
from jax.experimental.pallas import tpu as pltpu
from jax.experimental.pallas import tpu_sc as plsc

N_NODES = 50000
NUM_SEGS = 512
DIM = 256
SPAD = 544
NC, NS = 2, 16
SEGW = NUM_SEGS // NS
COLW = DIM // NC
T = 256
TK = 1024
NBLK = -(-N_NODES // TK)


def _tc_starts(ids_ref, out_ref):
    pid = pl.program_id(0)

    @pl.when(pid == 0)
    def _():
        out_ref[...] = jnp.zeros((1, SPAD), jnp.int32)

    rows = lax.broadcasted_iota(jnp.int32, (TK, 1), 0) + pid * TK
    segs = lax.broadcasted_iota(jnp.int32, (1, SPAD), 1)
    m = (ids_ref[...] < segs) & (rows < N_NODES)
    out_ref[...] += jnp.sum(m.astype(jnp.int32), axis=0, keepdims=True)


def _sc_pool(nodes_hbm, starts_hbm, sums_hbm, cnts_hbm, buf_v, acc_v, st_s, cnt_v):
    sid = lax.axis_index("s")
    cid = lax.axis_index("c")
    seg0 = sid * SEGW
    col0 = cid * COLW

    pltpu.sync_copy(starts_hbm, st_s)

    def _z(i, _):
        def _zc(j, _):
            acc_v[i, pl.ds(j * 16, 16)] = jnp.zeros((16,), jnp.float32)
            return 0
        return lax.fori_loop(0, COLW // 16, _zc, 0)
    lax.fori_loop(0, SEGW, _z, 0)

    row_lo = st_s[pl.ds(seg0, 16)][0]
    row_hi = st_s[pl.ds(seg0 + SEGW, 16)][0]
    w0 = pl.multiple_of((row_lo // 8) * 8, 8)
    n_win = (row_hi - w0 + T - 1) // T

    def _win(j, _):
        w = w0 + j * T
        wc = pl.multiple_of(jnp.minimum(w, N_NODES - T), 8)
        pltpu.sync_copy(
            nodes_hbm.at[pl.ds(wc, T), pl.ds(col0, COLW)], buf_v)

        def _seg(k, _):
            a = st_s[pl.ds(seg0 + k, 16)][0]
            b = st_s[pl.ds(seg0 + k + 1, 16)][0]
            lo = jnp.maximum(a, w) - wc
            hi = jnp.minimum(b, w + T) - wc

            def _row(r, accs):
                return tuple(accs[j2] + buf_v[r, pl.ds(j2 * 16, 16)]
                             for j2 in range(COLW // 16))
            accs0 = tuple(jnp.zeros((16,), jnp.float32)
                          for _ in range(COLW // 16))
            accs = lax.fori_loop(lo, hi, _row, accs0)

            @pl.when(hi > lo)
            def _():
                for j2 in range(COLW // 16):
                    plsc.addupdate(acc_v.at[k, pl.ds(j2 * 16, 16)], accs[j2])
            return 0

        lax.fori_loop(0, SEGW, _seg, 0)
        return 0

    lax.fori_loop(0, n_win, _win, 0)

    pltpu.sync_copy(acc_v, sums_hbm.at[pl.ds(seg0, SEGW), pl.ds(col0, COLW)])

    @pl.when(cid == 0)
    def _():
        for v in range(SEGW // 16):
            lo16 = st_s[pl.ds(seg0 + v * 16, 16)]
            hi16 = st_s[pl.ds(seg0 + v * 16 + 1, 16)]
            cnt_v[pl.ds(v * 16, 16)] = hi16 - lo16
        pltpu.sync_copy(cnt_v, cnts_hbm.at[pl.ds(seg0, SEGW)])


def _tc_finish(ps_ref, pc_ref, w1_ref, b1_ref, w2_ref, b2_ref, g_ref, z_ref):
    counts = jnp.maximum(pc_ref[...].astype(jnp.float32), 1.0)
    g = ps_ref[...] / counts
    g_ref[...] = g
    h = lax.dot_general(g, w1_ref[...], (((1,), (1,)), ((), ())),
                        preferred_element_type=jnp.float32) + b1_ref[...]
    h = jnp.maximum(h, 0.0)
    z_ref[...] = lax.dot_general(h, w2_ref[...], (((1,), (1,)), ((), ())),
                                 preferred_element_type=jnp.float32) + b2_ref[...]


def kernel(node_rep, batch_ids, W1, b1, W2, b2):
    ids32 = batch_ids.astype(jnp.int32).reshape(N_NODES, 1)

    starts2d = pl.pallas_call(
        _tc_starts,
        grid=(NBLK,),
        in_specs=[pl.BlockSpec((TK, 1), lambda i: (i, 0))],
        out_specs=pl.BlockSpec((1, SPAD), lambda i: (0, 0)),
        out_shape=jax.ShapeDtypeStruct((1, SPAD), jnp.int32),
    )(ids32)
    starts = starts2d.reshape(SPAD)

    mesh = plsc.VectorSubcoreMesh(core_axis_name="c", subcore_axis_name="s",
                                  num_cores=NC, num_subcores=NS)
    sums, cnts = pl.kernel(
        _sc_pool,
        out_type=(jax.ShapeDtypeStruct((NUM_SEGS, DIM), jnp.float32),
                  jax.ShapeDtypeStruct((NUM_SEGS,), jnp.int32)),
        mesh=mesh,
        scratch_types=[
            pltpu.VMEM((T, COLW), jnp.float32),
            pltpu.VMEM((SEGW, COLW), jnp.float32),
            pltpu.VMEM((SPAD,), jnp.int32),
            pltpu.VMEM((SEGW,), jnp.int32),
        ],
    )(node_rep, starts)

    g, z = pl.pallas_call(
        _tc_finish,
        out_shape=(jax.ShapeDtypeStruct((NUM_SEGS, DIM), jnp.float32),
                   jax.ShapeDtypeStruct((NUM_SEGS, DIM), jnp.float32)),
    )(sums, cnts.reshape(NUM_SEGS, 1), W1, b1.reshape(1, DIM),
      W2, b2.reshape(1, DIM))

    return (g, z)


# speedup vs baseline: 5.4364x; 1.5522x over previous
import jax
import jax.numpy as jnp
from jax import lax
from jax.experimental import pallas as pl
from jax.experimental.pallas import tpu as pltpu
from jax.experimental.pallas import tpu_sc as plsc

N_NODES = 50000
NUM_SEGS = 512
DIM = 256
NC, NS = 2, 16
SEGW = NUM_SEGS // NS
COLW = DIM // NC
NV = COLW // 16
T = 256
IDPAD = N_NODES + 16
STW = 64


def _sc_pool(nodes_hbm, ids_hbm, sums_hbm, cnts_hbm,
             buf_v, acc_v, ids_v, st_s, cnt_v):
    sid = lax.axis_index("s")
    cid = lax.axis_index("c")
    seg0 = sid * SEGW
    col0 = cid * COLW

    pltpu.sync_copy(ids_hbm, ids_v.at[pl.ds(0, N_NODES)])

    def _z(i, _):
        def _zc(j, _):
            acc_v[i, pl.ds(j * 16, 16)] = jnp.zeros((16,), jnp.float32)
            return 0
        return lax.fori_loop(0, NV, _zc, 0)
    lax.fori_loop(0, SEGW, _z, 0)

    lane = lax.iota(jnp.int32, 16)

    def _bnd(m, vecs):
        target = seg0 + m

        def _bs(_, lh):
            lo, hi = lh
            mid = (lo + hi) // 2
            v = ids_v[pl.ds(mid, 16)][0]
            lt = v < target
            return (jnp.where(lt, mid + 1, lo), jnp.where(lt, hi, mid))

        lo, _ = lax.fori_loop(0, 16, _bs,
                              (jnp.int32(0), jnp.int32(N_NODES)))
        v0, v1, v2 = vecs
        v0 = jnp.where(lane == m, lo, v0)
        v1 = jnp.where(lane == m - 16, lo, v1)
        v2 = jnp.where(lane == m - 32, lo, v2)
        return (v0, v1, v2)

    z16 = jnp.zeros((16,), jnp.int32)
    v0, v1, v2 = lax.fori_loop(0, SEGW + 1, _bnd, (z16, z16, z16))
    st_s[pl.ds(0, 16)] = v0
    st_s[pl.ds(16, 16)] = v1
    st_s[pl.ds(32, 16)] = v2

    row_lo = st_s[pl.ds(0, 16)][0]
    row_hi = st_s[pl.ds(SEGW, 16)][0]
    w0 = pl.multiple_of((row_lo // 8) * 8, 8)
    n_win = (row_hi - w0 + T - 1) // T

    def _win(j, _):
        w = w0 + j * T
        wc = pl.multiple_of(jnp.minimum(w, N_NODES - T), 8)
        pltpu.sync_copy(
            nodes_hbm.at[pl.ds(wc, T), pl.ds(col0, COLW)], buf_v)

        def _seg(k, _):
            a = st_s[pl.ds(k, 16)][0]
            b = st_s[pl.ds(k + 1, 16)][0]
            lo = jnp.maximum(a, w) - wc
            hi = jnp.minimum(b, w + T) - wc

            def _row(r, accs):
                return tuple(accs[j2] + buf_v[r, pl.ds(j2 * 16, 16)]
                             for j2 in range(NV))
            accs0 = tuple(jnp.zeros((16,), jnp.float32) for _ in range(NV))
            accs = lax.fori_loop(lo, hi, _row, accs0)

            @pl.when(hi > lo)
            def _():
                for j2 in range(NV):
                    plsc.addupdate(acc_v.at[k, pl.ds(j2 * 16, 16)], accs[j2])
            return 0

        lax.fori_loop(0, SEGW, _seg, 0)
        return 0

    lax.fori_loop(0, n_win, _win, 0)

    pltpu.sync_copy(acc_v, sums_hbm.at[pl.ds(seg0, SEGW), pl.ds(col0, COLW)])

    @pl.when(cid == 0)
    def _():
        for v in range(SEGW // 16):
            lo16 = st_s[pl.ds(v * 16, 16)]
            hi16 = st_s[pl.ds(v * 16 + 1, 16)]
            cnt_v[pl.ds(v * 16, 16)] = hi16 - lo16
        pltpu.sync_copy(cnt_v, cnts_hbm.at[pl.ds(seg0, SEGW)])


def _tc_finish(ps_ref, pc_ref, w1_ref, b1_ref, w2_ref, b2_ref, g_ref, z_ref):
    counts = jnp.maximum(pc_ref[...].astype(jnp.float32), 1.0)
    g = ps_ref[...] / counts
    g_ref[...] = g
    h = lax.dot_general(g, w1_ref[...], (((1,), (1,)), ((), ())),
                        preferred_element_type=jnp.float32) + b1_ref[...]
    h = jnp.maximum(h, 0.0)
    z_ref[...] = lax.dot_general(h, w2_ref[...], (((1,), (1,)), ((), ())),
                                 preferred_element_type=jnp.float32) + b2_ref[...]


def kernel(node_rep, batch_ids, W1, b1, W2, b2):
    ids32 = batch_ids.astype(jnp.int32)

    mesh = plsc.VectorSubcoreMesh(core_axis_name="c", subcore_axis_name="s",
                                  num_cores=NC, num_subcores=NS)
    sums, cnts = pl.kernel(
        _sc_pool,
        out_type=(jax.ShapeDtypeStruct((NUM_SEGS, DIM), jnp.float32),
                  jax.ShapeDtypeStruct((NUM_SEGS,), jnp.int32)),
        mesh=mesh,
        scratch_types=[
            pltpu.VMEM((T, COLW), jnp.float32),
            pltpu.VMEM((SEGW, COLW), jnp.float32),
            pltpu.VMEM((IDPAD,), jnp.int32),
            pltpu.VMEM((STW,), jnp.int32),
            pltpu.VMEM((SEGW,), jnp.int32),
        ],
    )(node_rep, ids32)

    g, z = pl.pallas_call(
        _tc_finish,
        out_shape=(jax.ShapeDtypeStruct((NUM_SEGS, DIM), jnp.float32),
                   jax.ShapeDtypeStruct((NUM_SEGS, DIM), jnp.float32)),
    )(sums, cnts.reshape(NUM_SEGS, 1), W1, b1.reshape(1, DIM),
      W2, b2.reshape(1, DIM))

    return (g, z)
